# baseline (speedup 1.0000x reference)
import functools

import jax
import jax.numpy as jnp
from jax import lax
from jax.experimental import pallas as pl
from jax.experimental.pallas import tpu as pltpu
from jax.experimental.pallas import tpu_sc as plsc

_LANES = 16
_NC = 2
_NS = 16
_NW = _NC * _NS

_GDN = lax.GatherDimensionNumbers(
    offset_dims=(), collapsed_slice_dims=(0,), start_index_map=(0,)
)


def _vsplat(vec, sel):
    return lax.gather(
        vec,
        sel[:, None],
        dimension_numbers=_GDN,
        slice_sizes=(1,),
        mode=lax.GatherScatterMode.PROMISE_IN_BOUNDS,
    )


def _make_sc_pool(B, LP, EMB):
    bags_per_w = B // _NW
    chunk = 8
    nchunks = bags_per_w // chunk
    nsub = EMB // _LANES
    cw = chunk * LP

    mesh = plsc.VectorSubcoreMesh(core_axis_name="c", subcore_axis_name="s")

    @functools.partial(
        pl.kernel,
        mesh=mesh,
        compiler_params=pltpu.CompilerParams(
            needs_layout_passes=False, use_tc_tiling_on_sc=False
        ),
        out_type=jax.ShapeDtypeStruct((B, EMB), jnp.float32),
        scratch_types=[
            pltpu.VMEM((chunk, LP), jnp.int32),
            pltpu.VMEM((chunk, LP), jnp.int32),
            pltpu.VMEM((chunk, LP), jnp.int32),
            pltpu.VMEM((cw, 2 * EMB), jnp.float32),
            pltpu.VMEM((bags_per_w, EMB), jnp.float32),
            pltpu.SemaphoreType.DMA,
        ],
    )
    def sc_pool(data_hbm, table_hbm, out_hbm, idx_v, hi_v, off_v, rows_v,
                pooled_v, sem):
        wid = lax.axis_index("s") * _NC + lax.axis_index("c")
        bag0 = wid * bags_per_w
        lane = lax.iota(jnp.int32, _LANES)
        cols = [j * _LANES + lane for j in range(nsub)]
        lane_sel = [jnp.full((_LANES,), u, jnp.int32) for u in range(_LANES)]

        def chunk_body(gc, carry):
            row0 = bag0 + gc * chunk
            pltpu.sync_copy(data_hbm.at[pl.ds(row0, chunk), :], idx_v)
            for i in range(chunk):
                for k in range(LP // _LANES):
                    v = idx_v[i, pl.ds(k * _LANES, _LANES)]
                    hi_v[i, pl.ds(k * _LANES, _LANES)] = v >> 1
                    off_v[i, pl.ds(k * _LANES, _LANES)] = (v & 1) << 6
            copies = [
                pltpu.async_copy(
                    table_hbm.at[hi_v.at[i]],
                    rows_v.at[pl.ds(i * LP, LP), :],
                    sem,
                )
                for i in range(chunk)
            ]
            for c in copies:
                c.wait()
            for i in range(chunk):
                def accum(g, accs):
                    offg = off_v[i, pl.ds(g * _LANES, _LANES)]
                    out = list(accs)
                    for u in range(_LANES):
                        rowsplat = jnp.full(
                            (_LANES,), i * LP + g * _LANES + u, jnp.int32
                        )
                        offsplat = _vsplat(offg, lane_sel[u])
                        for j in range(nsub):
                            out[j] = out[j] + plsc.load_gather(
                                rows_v, [rowsplat, offsplat + cols[j]]
                            )
                    return tuple(out)

                zero = jnp.zeros((_LANES,), jnp.float32)
                accs = lax.fori_loop(0, LP // _LANES, accum, (zero,) * nsub)
                for j in range(nsub):
                    pooled_v[gc * chunk + i, pl.ds(j * _LANES, _LANES)] = accs[j]
            return carry

        lax.fori_loop(0, nchunks, chunk_body, 0)
        pltpu.sync_copy(pooled_v, out_hbm.at[pl.ds(bag0, bags_per_w), :])

    return sc_pool


def _finalize(pooled, lenf, wt, b2):
    B, EMB = pooled.shape
    NCLS = wt.shape[1]

    def body(p_ref, l_ref, w_ref, b_ref, o_ref):
        x = p_ref[...] / l_ref[...]
        o_ref[...] = (
            jnp.dot(x, w_ref[...], preferred_element_type=jnp.float32) + b_ref[...]
        )

    return pl.pallas_call(
        body,
        out_shape=jax.ShapeDtypeStruct((B, NCLS), jnp.float32),
    )(pooled, lenf, wt, b2)


def kernel(data, length, embed_table, W, b):
    B, L = data.shape
    V, EMB = embed_table.shape
    NCLS = W.shape[0]
    LP = 64

    data_pad = jnp.concatenate(
        [data, jnp.zeros((B, LP - L), jnp.int32)], axis=1
    )
    table2 = embed_table.reshape(V // 2, 2 * EMB)
    pooled = _make_sc_pool(B, LP, EMB)(data_pad, table2)
    lenf = length.astype(jnp.float32).reshape(B, 1)
    return _finalize(pooled, lenf, W.T, b.reshape(1, NCLS))


# speedup vs baseline: 4.3089x; 4.3089x over previous
import functools

import jax
import jax.numpy as jnp
from jax import lax
from jax.experimental import pallas as pl
from jax.experimental.pallas import tpu as pltpu
from jax.experimental.pallas import tpu_sc as plsc

_LANES = 16
_NC = 2
_NS = 16
_NW = _NC * _NS


def _make_sc_pool(B, L, LP, EMB):
    bags_per_w = B // _NW
    chunk = 8
    nchunks = bags_per_w // chunk
    nsub = EMB // _LANES
    cr = chunk * LP
    full_groups = L // _LANES
    rem = L - full_groups * _LANES

    mesh = plsc.VectorSubcoreMesh(core_axis_name="c", subcore_axis_name="s")

    @functools.partial(
        pl.kernel,
        mesh=mesh,
        compiler_params=pltpu.CompilerParams(
            needs_layout_passes=False, use_tc_tiling_on_sc=False
        ),
        out_type=jax.ShapeDtypeStruct((B, EMB), jnp.float32),
        scratch_types=[
            pltpu.VMEM((chunk, LP), jnp.int32),
            pltpu.VMEM((chunk, LP), jnp.int32),
            pltpu.VMEM((cr, EMB), jnp.float32),
            pltpu.VMEM((cr, EMB), jnp.float32),
            pltpu.VMEM((bags_per_w, EMB), jnp.float32),
            pltpu.SemaphoreType.DMA,
            pltpu.SemaphoreType.DMA,
        ],
    )
    def sc_pool(data_hbm, table_hbm, out_hbm, idx0_v, idx1_v, rows0_v,
                rows1_v, pooled_v, sem0, sem1):
        wid = lax.axis_index("s") * _NC + lax.axis_index("c")
        bag0 = wid * bags_per_w
        idx_bufs = (idx0_v, idx1_v)
        row_bufs = (rows0_v, rows1_v)
        sems = (sem0, sem1)

        def fire(gc, buf):
            pltpu.sync_copy(
                data_hbm.at[pl.ds(bag0 + gc * chunk, chunk), :], idx_bufs[buf]
            )
            for i in range(chunk):
                pltpu.async_copy(
                    table_hbm.at[idx_bufs[buf].at[i]],
                    row_bufs[buf].at[pl.ds(i * LP, LP), :],
                    sems[buf],
                )

        def drain(buf):
            pltpu.make_async_copy(
                table_hbm.at[pl.ds(0, cr), :], row_bufs[buf], sems[buf]
            ).wait()

        def accum_chunk(gc, buf):
            rows_v = row_bufs[buf]
            for i in range(chunk):
                def body(g, accs):
                    out = list(accs)
                    for u in range(_LANES):
                        tok = i * LP + g * _LANES + u
                        for j in range(nsub):
                            out[j] = out[j] + rows_v[
                                tok, pl.ds(j * _LANES, _LANES)
                            ]
                    return tuple(out)

                zero = jnp.zeros((_LANES,), jnp.float32)
                accs = lax.fori_loop(0, full_groups, body, (zero,) * nsub)
                accs = list(accs)
                for u in range(rem):
                    tok = i * LP + full_groups * _LANES + u
                    for j in range(nsub):
                        accs[j] = accs[j] + rows_v[
                            tok, pl.ds(j * _LANES, _LANES)
                        ]
                for j in range(nsub):
                    pooled_v[gc * chunk + i, pl.ds(j * _LANES, _LANES)] = (
                        accs[j]
                    )

        fire(0, 0)

        def pair_body(k, carry):
            ga = 2 * k
            fire(ga + 1, 1)
            drain(0)
            accum_chunk(ga, 0)
            fire(jnp.minimum(ga + 2, nchunks - 1), 0)
            drain(1)
            accum_chunk(ga + 1, 1)
            return carry

        lax.fori_loop(0, nchunks // 2, pair_body, 0)
        drain(0)
        pltpu.sync_copy(pooled_v, out_hbm.at[pl.ds(bag0, bags_per_w), :])

    return sc_pool


def _finalize(pooled, lenf, wt, b2):
    B, EMB = pooled.shape
    NCLS = wt.shape[1]

    def body(p_ref, l_ref, w_ref, b_ref, o_ref):
        x = p_ref[...] / l_ref[...]
        o_ref[...] = (
            jnp.dot(x, w_ref[...], preferred_element_type=jnp.float32) + b_ref[...]
        )

    return pl.pallas_call(
        body,
        out_shape=jax.ShapeDtypeStruct((B, NCLS), jnp.float32),
    )(pooled, lenf, wt, b2)


def kernel(data, length, embed_table, W, b):
    B, L = data.shape
    V, EMB = embed_table.shape
    NCLS = W.shape[0]
    LP = 56

    pad_block = (
        jnp.arange(B, dtype=jnp.int32)[:, None] * (LP - L)
        + jnp.arange(LP - L, dtype=jnp.int32)[None, :]
    ) % V
    data_pad = jnp.concatenate([data, pad_block], axis=1)
    pooled = _make_sc_pool(B, L, LP, EMB)(data_pad, embed_table)
    lenf = length.astype(jnp.float32).reshape(B, 1)
    return _finalize(pooled, lenf, W.T, b.reshape(1, NCLS))
